# Initial kernel scaffold; baseline (speedup 1.0000x reference)
import jax
import jax.numpy as jnp
from jax.experimental import pallas as pl


def kernel(coeffs, coeffs_derivs, central_atom_index, neigh_atom_index, w1, b1, w_last, b_last):
    raise NotImplementedError("write your pallas kernel here")


# speedup vs baseline: 1.5041x; 1.5041x over previous
import functools

import jax
import jax.numpy as jnp
from jax import lax
from jax.experimental import pallas as pl
from jax.experimental.pallas import tpu as pltpu
from jax.experimental.pallas import tpu_sc as plsc

NA = 2048
NP = 65536
D = 64
K = 3
NC = 2
NS = 16
NW = NC * NS
PPW = NP // NW
S = 128
NSUB = PPW // S
FLAT = K * NA
STRIPE = FLAT // NS

_GATHER_DN = lax.GatherDimensionNumbers(
    offset_dims=(), collapsed_slice_dims=(0,), start_index_map=(0,))


def _take16(v, idx):
    return lax.gather(v, idx[:, None], _GATHER_DN, slice_sizes=(1,),
                      mode=lax.GatherScatterMode.PROMISE_IN_BOUNDS)


def _dense_body(x_ref, w1_ref, b1_ref, wl_ref, bl_ref, g_ref, e_ref):
    x = x_ref[...]
    w1 = w1_ref[...]
    wl = wl_ref[...]
    e_pa = jnp.tanh(x * w1 + b1_ref[...])
    g_ref[...] = (1.0 - e_pa * e_pa) * (w1 * wl)
    e_ref[0, 0] = jnp.sum(e_pa * wl) / float(NA) + bl_ref[0, 0]


def _dense_call(x, w1, b1, wl, bl):
    return pl.pallas_call(
        _dense_body,
        out_shape=[
            jax.ShapeDtypeStruct((NA, D), jnp.float32),
            jax.ShapeDtypeStruct((1, 1), jnp.float32),
        ],
        in_specs=[
            pl.BlockSpec(memory_space=pltpu.VMEM),
            pl.BlockSpec(memory_space=pltpu.VMEM),
            pl.BlockSpec(memory_space=pltpu.VMEM),
            pl.BlockSpec(memory_space=pltpu.VMEM),
            pl.BlockSpec(memory_space=pltpu.SMEM),
        ],
        out_specs=[
            pl.BlockSpec(memory_space=pltpu.VMEM),
            pl.BlockSpec(memory_space=pltpu.SMEM),
        ],
    )(x, w1, b1, wl, bl)


_mesh = plsc.VectorSubcoreMesh(core_axis_name="c", subcore_axis_name="s")


@functools.partial(
    pl.kernel,
    out_type=jax.ShapeDtypeStruct((NC, FLAT), jnp.float32),
    mesh=_mesh,
    compiler_params=pltpu.CompilerParams(
        needs_layout_passes=False, use_tc_tiling_on_sc=False),
    scratch_types=[
        pltpu.VMEM((S,), jnp.int32),
        pltpu.VMEM((S,), jnp.int32),
        pltpu.VMEM((S, D), jnp.float32),
        pltpu.VMEM((K, S, D), jnp.float32),
        pltpu.VMEM((FLAT,), jnp.float32),
        pltpu.VMEM((STRIPE,), jnp.float32),
        pltpu.VMEM((STRIPE,), jnp.float32),
        pltpu.VMEM_SHARED((NS, FLAT), jnp.float32),
        pltpu.SemaphoreType.DMA,
    ],
)
def _sc_pairs(g_hbm, cd_hbm, cen_hbm, nei_hbm, out_hbm,
              cidx, nidx, grows, cds, acc, stmp, sred, shared, sem):
    c = lax.axis_index("c")
    s = lax.axis_index("s")
    wid = s * NC + c
    iota = lax.iota(jnp.int32, 16)
    lane0 = iota == 0
    zeros16 = jnp.zeros((16,), jnp.float32)

    def _zero(i, carry):
        acc[pl.ds(i * 16, 16)] = zeros16
        return carry
    lax.fori_loop(0, FLAT // 16, _zero, 0)

    base0 = wid * PPW

    def _sub(sub, carry):
        base = base0 + sub * S
        pltpu.sync_copy(cen_hbm.at[pl.ds(base, S)], cidx)
        pltpu.sync_copy(nei_hbm.at[pl.ds(base, S)], nidx)
        pltpu.async_copy(g_hbm.at[cidx], grows, sem).wait()
        for k in range(K):
            pltpu.sync_copy(cd_hbm.at[k, pl.ds(base, S)], cds.at[k])

        def _batch(b, carry2):
            nvec = nidx[pl.ds(b * 16, 16)]
            for j in range(16):
                i = b * 16 + j
                nb = _take16(nvec, jnp.full((16,), j, jnp.int32))
                for k in range(K):
                    a = grows[i, pl.ds(0, 16)] * cds[k, i, pl.ds(0, 16)]
                    for q in range(1, 4):
                        a = a + grows[i, pl.ds(q * 16, 16)] * cds[k, i, pl.ds(q * 16, 16)]
                    for sh in (8, 4, 2, 1):
                        a = a + _take16(a, (iota + sh) & 15)
                    plsc.addupdate_scatter(acc, [nb + (k * NA)], -a, mask=lane0)
            return carry2
        lax.fori_loop(0, S // 16, _batch, 0)
        return carry
    lax.fori_loop(0, NSUB, _sub, 0)

    pltpu.sync_copy(acc, shared.at[s])
    plsc.subcore_barrier()

    def _zero2(i, carry):
        sred[pl.ds(i * 16, 16)] = zeros16
        return carry
    lax.fori_loop(0, STRIPE // 16, _zero2, 0)

    def _red(w, carry):
        pltpu.sync_copy(shared.at[w, pl.ds(s * STRIPE, STRIPE)], stmp)

        def _addv(i, carry2):
            sred[pl.ds(i * 16, 16)] = sred[pl.ds(i * 16, 16)] + stmp[pl.ds(i * 16, 16)]
            return carry2
        lax.fori_loop(0, STRIPE // 16, _addv, 0)
        return carry
    lax.fori_loop(0, NS, _red, 0)

    pltpu.sync_copy(sred, out_hbm.at[c, pl.ds(s * STRIPE, STRIPE)])


def kernel(coeffs, coeffs_derivs, central_atom_index, neigh_atom_index,
           w1, b1, w_last, b_last):
    x = coeffs.reshape(NA, D)
    cd = coeffs_derivs.reshape(K, NP, D)
    g_arr, e_arr = _dense_call(
        x,
        w1.reshape(1, D),
        b1.reshape(1, D),
        w_last.reshape(1, D),
        b_last.reshape(1, 1),
    )
    cen = central_atom_index.astype(jnp.int32)
    nei = neigh_atom_index.astype(jnp.int32)
    parts = _sc_pairs(g_arr, cd, cen, nei)
    out_f = (parts[0] + parts[1]).reshape(1, K, NA)
    e_out = e_arr.reshape(1)
    return (e_out, out_f)
